# baseline (speedup 1.0000x reference)
import functools

import jax
import jax.numpy as jnp
import numpy as np
from jax import lax
from jax.experimental import pallas as pl
from jax.experimental.pallas import tpu as pltpu
from jax.experimental.pallas import tpu_sc as plsc

N = 10000
E = 320000
CIN = 128
HID = 64
BN = 400
NB = N // BN
NW = 32
EPT = E // NW
E2 = E // 2
EP2 = 163840
BE = 8192
EPT2 = E2 // NW
CHG = 40
NCG = EPT2 // CHG
GRPG = 5
NGG = NCG // GRPG
GCHG = GRPG * CHG
CHUNK = 80
NCHUNK = EPT // CHUNK
GRP = 5
NGRP = NCHUNK // GRP
GCH = GRP * CHUNK
SCH = 40
SNC = EPT // SCH
SGRP = 2
SNG = SNC // SGRP
SGCH = SGRP * SCH
NPAD = 10240
TRASH = N
LN_EPS = 1e-5

_SC_PARAMS = pltpu.CompilerParams(needs_layout_passes=False,
                                  use_tc_tiling_on_sc=False)
_SC_MESH = dict(core_axis_name="c", subcore_axis_name="s")


def _logit_thresholds() -> np.ndarray:
    def rotl(x, d):
        return ((x << np.uint32(d)) | (x >> np.uint32(32 - d))).astype(np.uint32)

    R = [13, 15, 26, 6, 17, 29, 16, 24]
    ks = [np.uint32(0), np.uint32(42), np.uint32(0 ^ 42 ^ 0x1BD11BDA)]
    x0 = np.zeros(E, dtype=np.uint32) + ks[0]
    x1 = (np.arange(E, dtype=np.uint32) + ks[1]).astype(np.uint32)
    for i in range(5):
        for r in range(4):
            x0 = (x0 + x1).astype(np.uint32)
            x1 = rotl(x1, R[(i % 2) * 4 + r])
            x1 = (x1 ^ x0).astype(np.uint32)
        x0 = (x0 + ks[(i + 1) % 3]).astype(np.uint32)
        x1 = (x1 + ks[(i + 2) % 3] + np.uint32(i + 1)).astype(np.uint32)
    bits = x0 ^ x1
    u = ((bits >> np.uint32(9)) | np.uint32(0x3F800000)).view(np.float32) \
        - np.float32(1.0)
    return np.log(u) - np.log1p(-u)


_THR_FULL = _logit_thresholds()
_PADH = np.zeros(EP2 - E2, np.float32)
_THR_A = np.concatenate([_THR_FULL[:E2], _PADH]).reshape(EP2 // 128, 128)
_THR_B = np.concatenate([_THR_FULL[E2:], _PADH]).reshape(EP2 // 128, 128)


def _sc_gather_body(row_h, col_h, x_h, xi_h, xj_h,
                    ridx_v, cidx_v, bufi, bufj, semgi, semgj, semwi, semwj):
    cid = lax.axis_index("c")
    sid = lax.axis_index("s")
    ebase = (cid * 16 + sid) * EPT2

    pltpu.sync_copy(row_h.at[pl.ds(ebase, EPT2)], ridx_v)
    pltpu.sync_copy(col_h.at[pl.ds(ebase, EPT2)], cidx_v)

    def grp_body(g, carry):
        @pl.when(g > 0)
        def _():
            pltpu.make_async_copy(bufi, xi_h.at[pl.ds(0, GCHG), :], semwi).wait()
            pltpu.make_async_copy(bufj, xj_h.at[pl.ds(0, GCHG), :], semwj).wait()
        for b in range(GRPG):
            off = g * GCHG + b * CHG
            pltpu.async_copy(x_h.at[ridx_v.at[pl.ds(off, CHG)]],
                             bufi.at[pl.ds(b * CHG, CHG), :], semgi)
            pltpu.async_copy(x_h.at[cidx_v.at[pl.ds(off, CHG)]],
                             bufj.at[pl.ds(b * CHG, CHG), :], semgj)
        pltpu.make_async_copy(x_h.at[pl.ds(0, GCHG), :], bufi, semgi).wait()
        pltpu.make_async_copy(x_h.at[pl.ds(0, GCHG), :], bufj, semgj).wait()
        base = ebase + g * GCHG
        pltpu.async_copy(bufi, xi_h.at[pl.ds(base, GCHG), :], semwi)
        pltpu.async_copy(bufj, xj_h.at[pl.ds(base, GCHG), :], semwj)
        return carry

    lax.fori_loop(0, NGG, grp_body, 0)
    pltpu.make_async_copy(bufi, xi_h.at[pl.ds(0, GCHG), :], semwi).wait()
    pltpu.make_async_copy(bufj, xj_h.at[pl.ds(0, GCHG), :], semwj).wait()


def _sc_gather(row, col, x):
    f = functools.partial(
        pl.kernel,
        out_type=[
            jax.ShapeDtypeStruct((EP2, CIN), jnp.float32),
            jax.ShapeDtypeStruct((EP2, CIN), jnp.float32),
        ],
        mesh=plsc.VectorSubcoreMesh(**_SC_MESH),
        scratch_types=[
            pltpu.VMEM((EPT2,), jnp.int32),
            pltpu.VMEM((EPT2,), jnp.int32),
            pltpu.VMEM((GCHG, CIN), jnp.float32),
            pltpu.VMEM((GCHG, CIN), jnp.float32),
            pltpu.SemaphoreType.DMA,
            pltpu.SemaphoreType.DMA,
            pltpu.SemaphoreType.DMA,
            pltpu.SemaphoreType.DMA,
        ],
        compiler_params=_SC_PARAMS,
    )(_sc_gather_body)
    return f(row, col, x)


def _tc_score_body(xi_ref, xj_ref, col_ref, thr_ref, g_ref, be_ref,
                   w1t_ref, b1_ref, w2_ref, b2_ref, colr_ref):
    xi = xi_ref[...]
    xj = xj_ref[...]
    s = jnp.sum(xi, axis=-1, keepdims=True) + jnp.sum(xj, axis=-1, keepdims=True)
    mu = s * (1.0 / (2 * CIN))
    di = xi - mu
    dj = xj - mu
    s2 = (jnp.sum(di * di, axis=-1, keepdims=True)
          + jnp.sum(dj * dj, axis=-1, keepdims=True))
    var = s2 * (1.0 / (2 * CIN))
    rstd = 1.0 / jnp.sqrt(var + LN_EPS)
    xni = di * rstd * g_ref[:, :CIN] + be_ref[:, :CIN]
    xnj = dj * rstd * g_ref[:, CIN:] + be_ref[:, CIN:]
    pre = (jnp.dot(xni, w1t_ref[:CIN, :], preferred_element_type=jnp.float32)
           + jnp.dot(xnj, w1t_ref[CIN:, :], preferred_element_type=jnp.float32))
    h = jnp.maximum(pre + b1_ref[...], 0.0)
    tot = jnp.dot(h, w2_ref[...], preferred_element_type=jnp.float32)
    totr = (tot + b2_ref[...]).reshape(BE // 128, 128)
    colr_ref[...] = jnp.where(totr > thr_ref[...], col_ref[...], TRASH)


def _tc_score(xi, xj, col2, thr2, gamma2, beta2, w1t, b1r, w2c, b2s):
    neb = xi.shape[0] // BE
    return pl.pallas_call(
        _tc_score_body,
        grid=(neb,),
        in_specs=[
            pl.BlockSpec((BE, CIN), lambda i: (i, 0)),
            pl.BlockSpec((BE, CIN), lambda i: (i, 0)),
            pl.BlockSpec((BE // 128, 128), lambda i: (i, 0)),
            pl.BlockSpec((BE // 128, 128), lambda i: (i, 0)),
            pl.BlockSpec((1, 2 * CIN), lambda i: (0, 0)),
            pl.BlockSpec((1, 2 * CIN), lambda i: (0, 0)),
            pl.BlockSpec((2 * CIN, HID), lambda i: (0, 0)),
            pl.BlockSpec((1, HID), lambda i: (0, 0)),
            pl.BlockSpec((HID, 1), lambda i: (0, 0)),
            pl.BlockSpec((1, 1), lambda i: (0, 0)),
        ],
        out_specs=pl.BlockSpec((BE // 128, 128), lambda i: (i, 0)),
        out_shape=jax.ShapeDtypeStruct((xi.shape[0] // 128, 128), jnp.int32),
    )(xi, xj, col2, thr2, gamma2, beta2, w1t, b1r, w2c, b2s)


def _sc_deg_body(colr_h, degp_h, cidx_v, deg_v, deg_all, dredw, dsl_v):
    cid = lax.axis_index("c")
    sid = lax.axis_index("s")
    ebase = (cid * 16 + sid) * EPT

    pltpu.sync_copy(colr_h.at[pl.ds(ebase, EPT)], cidx_v)
    z16 = jnp.zeros((16,), jnp.float32)
    for i in range(NPAD // 16):
        deg_v[pl.ds(i * 16, 16)] = z16
    ones16 = jnp.ones((16,), jnp.float32)

    def blk(i, carry):
        idx = cidx_v[pl.ds(i * 16, 16)]
        plsc.addupdate_scatter(deg_v, [idx], ones16)
        return carry

    lax.fori_loop(0, EPT // 16, blk, 0)
    pltpu.sync_copy(deg_v, deg_all.at[sid])
    plsc.subcore_barrier()
    pltpu.sync_copy(deg_all.at[:, pl.ds(sid * 640, 640)], dredw)
    for i in range(640 // 16):
        acc = dredw[0, pl.ds(i * 16, 16)]
        for r in range(1, 16):
            acc = acc + dredw[r, pl.ds(i * 16, 16)]
        dsl_v[pl.ds(i * 16, 16)] = acc
    pltpu.sync_copy(dsl_v, degp_h.at[pl.ds(cid * NPAD + sid * 640, 640)])


def _sc_deg(colr):
    f = functools.partial(
        pl.kernel,
        out_type=jax.ShapeDtypeStruct((2 * NPAD,), jnp.float32),
        mesh=plsc.VectorSubcoreMesh(**_SC_MESH),
        scratch_types=[
            pltpu.VMEM((EPT,), jnp.int32),
            pltpu.VMEM((NPAD,), jnp.float32),
            pltpu.VMEM_SHARED((16, NPAD), jnp.float32),
            pltpu.VMEM((16, 640), jnp.float32),
            pltpu.VMEM((640,), jnp.float32),
        ],
        compiler_params=_SC_PARAMS,
    )(_sc_deg_body)
    return f(colr)


def _tc_scale_body(x_ref, gw_ref, deg_ref, b_ref, xwd_ref, selfl_ref, dinv_ref):
    d = deg_ref[0] + deg_ref[1] + 1.0
    dinv = 1.0 / jnp.sqrt(d)
    xw = jnp.dot(x_ref[...], gw_ref[...], preferred_element_type=jnp.float32)
    xwd_ref[...] = xw * dinv
    selfl_ref[...] = xw * (dinv * dinv) + b_ref[...]
    dinv_ref[...] = dinv


def _tc_scale(x, gwt, deg3, b):
    return pl.pallas_call(
        _tc_scale_body,
        grid=(NB,),
        in_specs=[
            pl.BlockSpec((BN, CIN), lambda i: (i, 0)),
            pl.BlockSpec((CIN, CIN), lambda i: (0, 0)),
            pl.BlockSpec((2, BN, 1), lambda i: (0, i, 0)),
            pl.BlockSpec((1, CIN), lambda i: (0, 0)),
        ],
        out_specs=[
            pl.BlockSpec((BN, CIN), lambda i: (i, 0)),
            pl.BlockSpec((BN, CIN), lambda i: (i, 0)),
            pl.BlockSpec((BN, 1), lambda i: (i, 0)),
        ],
        out_shape=[
            jax.ShapeDtypeStruct((N, CIN), jnp.float32),
            jax.ShapeDtypeStruct((N, CIN), jnp.float32),
            jax.ShapeDtypeStruct((N, 1), jnp.float32),
        ],
    )(x, gwt, deg3, b)


def _sc_scatter_body(row_h, colr2_h, xwd_h, znode_h, accp_h,
                     acc_sh, ridx_v, cidx_v, rbuf0, rbuf1,
                     semg0, semg1, sems0, sems1):
    cid = lax.axis_index("c")
    sid = lax.axis_index("s")
    wid = cid * 16 + sid
    ebase = wid * EPT
    rbufs = (rbuf0, rbuf1)
    semgs = (semg0, semg1)
    semss = (sems0, sems1)

    pltpu.sync_copy(znode_h, acc_sh.at[pl.ds(sid * 640, 640)])
    pltpu.sync_copy(row_h.at[pl.ds(ebase, EPT)], ridx_v)
    pltpu.sync_copy(colr2_h.at[pl.ds(wid * SNC, SNC), :], cidx_v)
    plsc.subcore_barrier()

    def fire_g(g, st):
        for b in range(SGRP):
            off = g * SGCH + b * SCH
            pltpu.async_copy(xwd_h.at[ridx_v.at[pl.ds(off, SCH)]],
                             rbufs[st].at[pl.ds(b * SCH, SCH), :], semgs[st])

    def fire_s(g, st):
        for b in range(SGRP):
            pltpu.async_copy(rbufs[st].at[pl.ds(b * SCH, SCH), :],
                             acc_sh.at[cidx_v.at[g * SGRP + b]], semss[st],
                             add=True)

    def drain_g(st):
        pltpu.make_async_copy(xwd_h.at[pl.ds(0, SGCH), :], rbufs[st],
                              semgs[st]).wait()

    def drain_s(st):
        pltpu.make_async_copy(rbufs[st], acc_sh.at[pl.ds(0, SGCH)],
                              semss[st]).wait()

    fire_g(0, 0)

    def grp_body(g, carry):
        for st in range(2):
            @pl.when(g % 2 == st)
            def _():
                sn = 1 - st
                drain_g(st)

                @pl.when(g + 1 < SNG)
                def _():
                    @pl.when(g >= 1)
                    def _():
                        drain_s(sn)
                    fire_g(g + 1, sn)
                fire_s(g, st)
        return carry

    lax.fori_loop(0, SNG, grp_body, 0)
    drain_s(0)
    drain_s(1)
    plsc.subcore_barrier()
    pltpu.sync_copy(acc_sh.at[pl.ds(sid * 640, 640)],
                    accp_h.at[cid, pl.ds(sid * 640, 640), :])


def _sc_scatter(row, colr2, xwd, znode):
    f = functools.partial(
        pl.kernel,
        out_type=jax.ShapeDtypeStruct((2, NPAD, CIN), jnp.float32),
        mesh=plsc.VectorSubcoreMesh(**_SC_MESH),
        scratch_types=[
            pltpu.VMEM_SHARED((NPAD, CIN), jnp.float32),
            pltpu.VMEM((EPT,), jnp.int32),
            pltpu.VMEM((SNC, SCH), jnp.int32),
            pltpu.VMEM((SGCH, CIN), jnp.float32),
            pltpu.VMEM((SGCH, CIN), jnp.float32),
            pltpu.SemaphoreType.DMA,
            pltpu.SemaphoreType.DMA,
            pltpu.SemaphoreType.DMA,
            pltpu.SemaphoreType.DMA,
        ],
        compiler_params=_SC_PARAMS,
    )(_sc_scatter_body)
    return f(row, colr2, xwd, znode)


def _tc_final_body(acc_ref, dinv_ref, selfl_ref, out_ref):
    a = acc_ref[0] + acc_ref[1]
    out_ref[...] = a * dinv_ref[...] + selfl_ref[...]


def _tc_final(accp, dinv, selfl):
    return pl.pallas_call(
        _tc_final_body,
        grid=(NB,),
        in_specs=[
            pl.BlockSpec((2, BN, CIN), lambda i: (0, i, 0)),
            pl.BlockSpec((BN, 1), lambda i: (i, 0)),
            pl.BlockSpec((BN, CIN), lambda i: (i, 0)),
        ],
        out_specs=pl.BlockSpec((BN, CIN), lambda i: (i, 0)),
        out_shape=jax.ShapeDtypeStruct((N, CIN), jnp.float32),
    )(accp, dinv, selfl)


def kernel(x, edge_index, ln_gamma, ln_beta, sel_W1, sel_b1, sel_W2, sel_b2,
           gcn_W, gcn_b):
    row = edge_index[0]
    col = edge_index[1]

    zpad = jnp.zeros((EP2 - E2,), jnp.int32)
    wargs = (ln_gamma[None, :], ln_beta[None, :], sel_W1.T,
             sel_b1[None, :], sel_W2.T, sel_b2[None, :])
    xi_a, xj_a = _sc_gather(row[:E2], col[:E2], x)
    xi_b, xj_b = _sc_gather(row[E2:], col[E2:], x)
    colp_a = jnp.concatenate([col[:E2], zpad]).reshape(EP2 // 128, 128)
    colp_b = jnp.concatenate([col[E2:], zpad]).reshape(EP2 // 128, 128)
    colr_a = _tc_score(xi_a, xj_a, colp_a, jnp.asarray(_THR_A), *wargs)
    colr_b = _tc_score(xi_b, xj_b, colp_b, jnp.asarray(_THR_B), *wargs)
    colr = jnp.concatenate([colr_a.reshape(EP2)[:E2],
                            colr_b.reshape(EP2)[:E2]])
    znode = jnp.zeros((640, CIN), jnp.float32)
    degp = _sc_deg(colr)
    deg3 = jnp.stack([degp[:N], degp[NPAD:NPAD + N]]).reshape(2, N, 1)
    xwd, selfl, dinv = _tc_scale(x, gcn_W.T, deg3, gcn_b[None, :])
    accp = _sc_scatter(row, colr.reshape(E // SCH, SCH), xwd, znode)
    return _tc_final(accp, dinv, selfl)


# speedup vs baseline: 1.0857x; 1.0857x over previous
import functools

import jax
import jax.numpy as jnp
import numpy as np
from jax import lax
from jax.experimental import pallas as pl
from jax.experimental.pallas import tpu as pltpu
from jax.experimental.pallas import tpu_sc as plsc

N = 10000
E = 320000
CIN = 128
HID = 64
BN = 400
NB = N // BN
NW = 32
EPT = E // NW
E2 = E // 2
EP2 = 163840
BE = 8192
EPT2 = E2 // NW
CHG = 40
NCG = EPT2 // CHG
GRPG = 5
NGG = NCG // GRPG
GCHG = GRPG * CHG
CHUNK = 80
NCHUNK = EPT // CHUNK
GRP = 5
NGRP = NCHUNK // GRP
GCH = GRP * CHUNK
SCH = 40
SNC = EPT // SCH
SGRP = 2
SNG = SNC // SGRP
SGCH = SGRP * SCH
NPAD = 10240
TRASH = N
LN_EPS = 1e-5

_SC_PARAMS = pltpu.CompilerParams(needs_layout_passes=False,
                                  use_tc_tiling_on_sc=False)
_SC_MESH = dict(core_axis_name="c", subcore_axis_name="s")


def _logit_thresholds() -> np.ndarray:
    def rotl(x, d):
        return ((x << np.uint32(d)) | (x >> np.uint32(32 - d))).astype(np.uint32)

    R = [13, 15, 26, 6, 17, 29, 16, 24]
    ks = [np.uint32(0), np.uint32(42), np.uint32(0 ^ 42 ^ 0x1BD11BDA)]
    x0 = np.zeros(E, dtype=np.uint32) + ks[0]
    x1 = (np.arange(E, dtype=np.uint32) + ks[1]).astype(np.uint32)
    for i in range(5):
        for r in range(4):
            x0 = (x0 + x1).astype(np.uint32)
            x1 = rotl(x1, R[(i % 2) * 4 + r])
            x1 = (x1 ^ x0).astype(np.uint32)
        x0 = (x0 + ks[(i + 1) % 3]).astype(np.uint32)
        x1 = (x1 + ks[(i + 2) % 3] + np.uint32(i + 1)).astype(np.uint32)
    bits = x0 ^ x1
    u = ((bits >> np.uint32(9)) | np.uint32(0x3F800000)).view(np.float32) \
        - np.float32(1.0)
    return np.log(u) - np.log1p(-u)


_THR_FULL = _logit_thresholds()
_PADH = np.zeros(EP2 - E2, np.float32)
_THR_A = np.concatenate([_THR_FULL[:E2], _PADH]).reshape(EP2 // 128, 128)
_THR_B = np.concatenate([_THR_FULL[E2:], _PADH]).reshape(EP2 // 128, 128)


def _sc_gather_body(row_h, col_h, x_h, xi_h, xj_h,
                    ridx_v, cidx_v, bufi, bufj, semgi, semgj, semwi, semwj):
    cid = lax.axis_index("c")
    sid = lax.axis_index("s")
    ebase = (cid * 16 + sid) * EPT2

    pltpu.sync_copy(row_h.at[pl.ds(ebase, EPT2)], ridx_v)
    pltpu.sync_copy(col_h.at[pl.ds(ebase, EPT2)], cidx_v)

    def grp_body(g, carry):
        @pl.when(g > 0)
        def _():
            pltpu.make_async_copy(bufi, xi_h.at[pl.ds(0, GCHG), :], semwi).wait()
            pltpu.make_async_copy(bufj, xj_h.at[pl.ds(0, GCHG), :], semwj).wait()
        for b in range(GRPG):
            off = g * GCHG + b * CHG
            pltpu.async_copy(x_h.at[ridx_v.at[pl.ds(off, CHG)]],
                             bufi.at[pl.ds(b * CHG, CHG), :], semgi)
            pltpu.async_copy(x_h.at[cidx_v.at[pl.ds(off, CHG)]],
                             bufj.at[pl.ds(b * CHG, CHG), :], semgj)
        pltpu.make_async_copy(x_h.at[pl.ds(0, GCHG), :], bufi, semgi).wait()
        pltpu.make_async_copy(x_h.at[pl.ds(0, GCHG), :], bufj, semgj).wait()
        base = ebase + g * GCHG
        pltpu.async_copy(bufi, xi_h.at[pl.ds(base, GCHG), :], semwi)
        pltpu.async_copy(bufj, xj_h.at[pl.ds(base, GCHG), :], semwj)
        return carry

    lax.fori_loop(0, NGG, grp_body, 0)
    pltpu.make_async_copy(bufi, xi_h.at[pl.ds(0, GCHG), :], semwi).wait()
    pltpu.make_async_copy(bufj, xj_h.at[pl.ds(0, GCHG), :], semwj).wait()


def _sc_gather(row, col, x):
    f = functools.partial(
        pl.kernel,
        out_type=[
            jax.ShapeDtypeStruct((EP2, CIN), jnp.float32),
            jax.ShapeDtypeStruct((EP2, CIN), jnp.float32),
        ],
        mesh=plsc.VectorSubcoreMesh(**_SC_MESH),
        scratch_types=[
            pltpu.VMEM((EPT2,), jnp.int32),
            pltpu.VMEM((EPT2,), jnp.int32),
            pltpu.VMEM((GCHG, CIN), jnp.float32),
            pltpu.VMEM((GCHG, CIN), jnp.float32),
            pltpu.SemaphoreType.DMA,
            pltpu.SemaphoreType.DMA,
            pltpu.SemaphoreType.DMA,
            pltpu.SemaphoreType.DMA,
        ],
        compiler_params=_SC_PARAMS,
    )(_sc_gather_body)
    return f(row, col, x)


def _tc_score_body(xi_ref, xj_ref, col_ref, thr_ref, g_ref, be_ref,
                   w1t_ref, b1_ref, w2_ref, b2_ref, colr_ref):
    v = jnp.concatenate([xi_ref[...], xj_ref[...]], axis=1)
    mu = jnp.mean(v, axis=-1, keepdims=True)
    var = jnp.mean((v - mu) ** 2, axis=-1, keepdims=True)
    xn = (v - mu) / jnp.sqrt(var + LN_EPS) * g_ref[...] + be_ref[...]
    h = jnp.maximum(jnp.dot(xn, w1t_ref[...],
                            preferred_element_type=jnp.float32) + b1_ref[...], 0.0)
    tot = jnp.dot(h, w2_ref[...], preferred_element_type=jnp.float32)
    totr = (tot + b2_ref[...]).reshape(BE // 128, 128)
    colr_ref[...] = jnp.where(totr > thr_ref[...], col_ref[...], TRASH)


def _tc_score(xi, xj, col2, thr2, gamma2, beta2, w1t, b1r, w2c, b2s):
    neb = xi.shape[0] // BE
    return pl.pallas_call(
        _tc_score_body,
        grid=(neb,),
        in_specs=[
            pl.BlockSpec((BE, CIN), lambda i: (i, 0)),
            pl.BlockSpec((BE, CIN), lambda i: (i, 0)),
            pl.BlockSpec((BE // 128, 128), lambda i: (i, 0)),
            pl.BlockSpec((BE // 128, 128), lambda i: (i, 0)),
            pl.BlockSpec((1, 2 * CIN), lambda i: (0, 0)),
            pl.BlockSpec((1, 2 * CIN), lambda i: (0, 0)),
            pl.BlockSpec((2 * CIN, HID), lambda i: (0, 0)),
            pl.BlockSpec((1, HID), lambda i: (0, 0)),
            pl.BlockSpec((HID, 1), lambda i: (0, 0)),
            pl.BlockSpec((1, 1), lambda i: (0, 0)),
        ],
        out_specs=pl.BlockSpec((BE // 128, 128), lambda i: (i, 0)),
        out_shape=jax.ShapeDtypeStruct((xi.shape[0] // 128, 128), jnp.int32),
    )(xi, xj, col2, thr2, gamma2, beta2, w1t, b1r, w2c, b2s)


def _sc_deg_body(colr_h, degp_h, cidx_v, deg_v, deg_all, dredw, dsl_v):
    cid = lax.axis_index("c")
    sid = lax.axis_index("s")
    ebase = (cid * 16 + sid) * EPT

    pltpu.sync_copy(colr_h.at[pl.ds(ebase, EPT)], cidx_v)
    z16 = jnp.zeros((16,), jnp.float32)
    for i in range(NPAD // 16):
        deg_v[pl.ds(i * 16, 16)] = z16
    ones16 = jnp.ones((16,), jnp.float32)

    def blk(i, carry):
        idx = cidx_v[pl.ds(i * 16, 16)]
        plsc.addupdate_scatter(deg_v, [idx], ones16)
        return carry

    lax.fori_loop(0, EPT // 16, blk, 0)
    pltpu.sync_copy(deg_v, deg_all.at[sid])
    plsc.subcore_barrier()
    pltpu.sync_copy(deg_all.at[:, pl.ds(sid * 640, 640)], dredw)
    for i in range(640 // 16):
        acc = dredw[0, pl.ds(i * 16, 16)]
        for r in range(1, 16):
            acc = acc + dredw[r, pl.ds(i * 16, 16)]
        dsl_v[pl.ds(i * 16, 16)] = acc
    pltpu.sync_copy(dsl_v, degp_h.at[pl.ds(cid * NPAD + sid * 640, 640)])


def _sc_deg(colr):
    f = functools.partial(
        pl.kernel,
        out_type=jax.ShapeDtypeStruct((2 * NPAD,), jnp.float32),
        mesh=plsc.VectorSubcoreMesh(**_SC_MESH),
        scratch_types=[
            pltpu.VMEM((EPT,), jnp.int32),
            pltpu.VMEM((NPAD,), jnp.float32),
            pltpu.VMEM_SHARED((16, NPAD), jnp.float32),
            pltpu.VMEM((16, 640), jnp.float32),
            pltpu.VMEM((640,), jnp.float32),
        ],
        compiler_params=_SC_PARAMS,
    )(_sc_deg_body)
    return f(colr)


def _tc_scale_body(x_ref, gw_ref, deg_ref, b_ref, xwd_ref, selfl_ref, dinv_ref):
    d = deg_ref[0] + deg_ref[1] + 1.0
    dinv = 1.0 / jnp.sqrt(d)
    xw = jnp.dot(x_ref[...], gw_ref[...], preferred_element_type=jnp.float32)
    xwd_ref[...] = xw * dinv
    selfl_ref[...] = xw * (dinv * dinv) + b_ref[...]
    dinv_ref[...] = dinv


def _tc_scale(x, gwt, deg3, b):
    return pl.pallas_call(
        _tc_scale_body,
        grid=(NB,),
        in_specs=[
            pl.BlockSpec((BN, CIN), lambda i: (i, 0)),
            pl.BlockSpec((CIN, CIN), lambda i: (0, 0)),
            pl.BlockSpec((2, BN, 1), lambda i: (0, i, 0)),
            pl.BlockSpec((1, CIN), lambda i: (0, 0)),
        ],
        out_specs=[
            pl.BlockSpec((BN, CIN), lambda i: (i, 0)),
            pl.BlockSpec((BN, CIN), lambda i: (i, 0)),
            pl.BlockSpec((BN, 1), lambda i: (i, 0)),
        ],
        out_shape=[
            jax.ShapeDtypeStruct((N, CIN), jnp.float32),
            jax.ShapeDtypeStruct((N, CIN), jnp.float32),
            jax.ShapeDtypeStruct((N, 1), jnp.float32),
        ],
    )(x, gwt, deg3, b)


def _sc_scatter_body(row_h, colr2_h, xwd_h, znode_h, accp_h,
                     acc_sh, ridx_v, cidx_v, rbuf0, rbuf1,
                     semg0, semg1, sems0, sems1):
    cid = lax.axis_index("c")
    sid = lax.axis_index("s")
    wid = cid * 16 + sid
    ebase = wid * EPT
    rbufs = (rbuf0, rbuf1)
    semgs = (semg0, semg1)
    semss = (sems0, sems1)

    pltpu.sync_copy(znode_h, acc_sh.at[pl.ds(sid * 640, 640)])
    pltpu.sync_copy(row_h.at[pl.ds(ebase, EPT)], ridx_v)
    pltpu.sync_copy(colr2_h.at[pl.ds(wid * SNC, SNC), :], cidx_v)
    plsc.subcore_barrier()

    def fire_g(g, st):
        for b in range(SGRP):
            off = g * SGCH + b * SCH
            pltpu.async_copy(xwd_h.at[ridx_v.at[pl.ds(off, SCH)]],
                             rbufs[st].at[pl.ds(b * SCH, SCH), :], semgs[st])

    def fire_s(g, st):
        for b in range(SGRP):
            pltpu.async_copy(rbufs[st].at[pl.ds(b * SCH, SCH), :],
                             acc_sh.at[cidx_v.at[g * SGRP + b]], semss[st],
                             add=True)

    def drain_g(st):
        pltpu.make_async_copy(xwd_h.at[pl.ds(0, SGCH), :], rbufs[st],
                              semgs[st]).wait()

    def drain_s(st):
        pltpu.make_async_copy(rbufs[st], acc_sh.at[pl.ds(0, SGCH)],
                              semss[st]).wait()

    fire_g(0, 0)

    def grp_body(g, carry):
        for st in range(2):
            @pl.when(g % 2 == st)
            def _():
                sn = 1 - st
                drain_g(st)

                @pl.when(g + 1 < SNG)
                def _():
                    @pl.when(g >= 1)
                    def _():
                        drain_s(sn)
                    fire_g(g + 1, sn)
                fire_s(g, st)
        return carry

    lax.fori_loop(0, SNG, grp_body, 0)
    drain_s(0)
    drain_s(1)
    plsc.subcore_barrier()
    pltpu.sync_copy(acc_sh.at[pl.ds(sid * 640, 640)],
                    accp_h.at[cid, pl.ds(sid * 640, 640), :])


def _sc_scatter(row, colr2, xwd, znode):
    f = functools.partial(
        pl.kernel,
        out_type=jax.ShapeDtypeStruct((2, NPAD, CIN), jnp.float32),
        mesh=plsc.VectorSubcoreMesh(**_SC_MESH),
        scratch_types=[
            pltpu.VMEM_SHARED((NPAD, CIN), jnp.float32),
            pltpu.VMEM((EPT,), jnp.int32),
            pltpu.VMEM((SNC, SCH), jnp.int32),
            pltpu.VMEM((SGCH, CIN), jnp.float32),
            pltpu.VMEM((SGCH, CIN), jnp.float32),
            pltpu.SemaphoreType.DMA,
            pltpu.SemaphoreType.DMA,
            pltpu.SemaphoreType.DMA,
            pltpu.SemaphoreType.DMA,
        ],
        compiler_params=_SC_PARAMS,
    )(_sc_scatter_body)
    return f(row, colr2, xwd, znode)


def _tc_final_body(acc_ref, dinv_ref, selfl_ref, out_ref):
    a = acc_ref[0] + acc_ref[1]
    out_ref[...] = a * dinv_ref[...] + selfl_ref[...]


def _tc_final(accp, dinv, selfl):
    return pl.pallas_call(
        _tc_final_body,
        grid=(NB,),
        in_specs=[
            pl.BlockSpec((2, BN, CIN), lambda i: (0, i, 0)),
            pl.BlockSpec((BN, 1), lambda i: (i, 0)),
            pl.BlockSpec((BN, CIN), lambda i: (i, 0)),
        ],
        out_specs=pl.BlockSpec((BN, CIN), lambda i: (i, 0)),
        out_shape=jax.ShapeDtypeStruct((N, CIN), jnp.float32),
    )(accp, dinv, selfl)


def kernel(x, edge_index, ln_gamma, ln_beta, sel_W1, sel_b1, sel_W2, sel_b2,
           gcn_W, gcn_b):
    row = edge_index[0]
    col = edge_index[1]

    zpad = jnp.zeros((EP2 - E2,), jnp.int32)
    wargs = (ln_gamma[None, :], ln_beta[None, :], sel_W1.T,
             sel_b1[None, :], sel_W2.T, sel_b2[None, :])
    xi_a, xj_a = _sc_gather(row[:E2], col[:E2], x)
    xi_b, xj_b = _sc_gather(row[E2:], col[E2:], x)
    colp_a = jnp.concatenate([col[:E2], zpad]).reshape(EP2 // 128, 128)
    colp_b = jnp.concatenate([col[E2:], zpad]).reshape(EP2 // 128, 128)
    colr_a = _tc_score(xi_a, xj_a, colp_a, jnp.asarray(_THR_A), *wargs)
    colr_b = _tc_score(xi_b, xj_b, colp_b, jnp.asarray(_THR_B), *wargs)
    colr = jnp.concatenate([colr_a.reshape(EP2)[:E2],
                            colr_b.reshape(EP2)[:E2]])
    znode = jnp.zeros((640, CIN), jnp.float32)
    degp = _sc_deg(colr)
    deg3 = jnp.stack([degp[:N], degp[NPAD:NPAD + N]]).reshape(2, N, 1)
    xwd, selfl, dinv = _tc_scale(x, gcn_W.T, deg3, gcn_b[None, :])
    accp = _sc_scatter(row, colr.reshape(E // SCH, SCH), xwd, znode)
    return _tc_final(accp, dinv, selfl)
